# baseline (speedup 1.0000x reference)
import functools

import jax
import jax.numpy as jnp
from jax import lax
from jax.experimental import pallas as pl
from jax.experimental.pallas import tpu as pltpu
from jax.experimental.pallas import tpu_sc as plsc

N = 8192
D = 8
K = 32
E = 100000
H1 = 512
H2 = 512

R = 128
NW = 32
EPAD = 100352
EPW = EPAD // NW


def _leaky(x):
    return jnp.where(x >= 0, x, 0.01 * x)


INF_I = 0x7FFFFFFF
NC = 64
CW = N // NC
NQ = 10


def _keys(pts_blk, pts_all):
    g = lax.dot_general(pts_blk, pts_all, (((1,), (1,)), ((), ())),
                        preferred_element_type=jnp.float32)
    sq_blk = jnp.sum(pts_blk * pts_blk, axis=1, keepdims=True)
    sq_all = jnp.sum(pts_all * pts_all, axis=1, keepdims=True)
    d2 = sq_blk + sq_all.T - 2.0 * g
    d2 = jnp.maximum(d2, 1e-12)
    return lax.bitcast_convert_type(d2, jnp.int32)


def _mlp(knn_i, w1_ref, b1_ref, w2_ref, b2_ref, w3_ref, b3_ref):
    knn = jnp.sqrt(lax.bitcast_convert_type(knn_i, jnp.float32))
    h = _leaky(jnp.dot(knn, w1_ref[:], preferred_element_type=jnp.float32)
               + b1_ref[:])
    h = _leaky(jnp.dot(h, w2_ref[:], preferred_element_type=jnp.float32)
               + b2_ref[:])
    f = jnp.dot(h, w3_ref[:], preferred_element_type=jnp.float32)
    return f[:, 0] + b3_ref[0, 0]


def _vertex_body_exact(pts_blk_ref, pts_all_ref, w1_ref, b1_ref, w2_ref,
                       b2_ref, w3_ref, b3_ref, out_ref):
    ki = _keys(pts_blk_ref[:], pts_all_ref[:])
    cols = lax.broadcasted_iota(jnp.int32, (R, N), 1)
    vals = []
    for k in range(K + 1):
        m = jnp.min(ki, axis=1, keepdims=True)
        if k > 0:
            vals.append(m)
        if k < K:
            first = jnp.min(jnp.where(ki == m, cols, N), axis=1,
                            keepdims=True)
            ki = jnp.where(cols == first, INF_I, ki)
    knn_i = jnp.concatenate(vals, axis=1)
    out_ref[0, 0, :] = _mlp(knn_i, w1_ref, b1_ref, w2_ref, b2_ref,
                            w3_ref, b3_ref)


def _vertex_body_fast(pts_blk_ref, pts_all_ref, w1_ref, b1_ref, w2_ref,
                      b2_ref, w3_ref, b3_ref, out_ref, flag_ref):
    ki3 = _keys(pts_blk_ref[:], pts_all_ref[:]).reshape(R, NC, CW)
    lane = lax.broadcasted_iota(jnp.int32, (R, NC, CW), 2)
    qs = []
    for q in range(NQ):
        mc = jnp.min(ki3, axis=2, keepdims=True)
        qs.append(jnp.transpose(mc, (0, 2, 1)))
        if q < NQ - 1:
            firstl = jnp.min(jnp.where(ki3 == mc, lane, CW), axis=2,
                             keepdims=True)
            ki3 = jnp.where(lane == firstl, INF_I, ki3)
    qarr = jnp.concatenate(qs, axis=1)
    qlast = qs[-1][:, 0, :]
    lin = (lax.broadcasted_iota(jnp.int32, (R, NQ, NC), 1) * NC
           + lax.broadcasted_iota(jnp.int32, (R, NQ, NC), 2))
    vals = []
    for k in range(K + 1):
        m2 = jnp.min(qarr, axis=1)
        m = jnp.min(m2, axis=1, keepdims=True)
        vals.append(m)
        if k < K:
            m3 = m[:, :, None]
            f2 = jnp.min(jnp.where(qarr == m3, lin, NQ * NC), axis=1)
            first = jnp.min(f2, axis=1, keepdims=True)[:, :, None]
            qarr = jnp.where(lin == first, INF_I, qarr)
    v33 = vals[-1]
    flag = jnp.any(qlast <= v33, axis=1)
    knn_i = jnp.concatenate(vals[1:], axis=1)
    out_ref[0, 0, :] = _mlp(knn_i, w1_ref, b1_ref, w2_ref, b2_ref,
                            w3_ref, b3_ref)
    flag_ref[0, 0, :] = flag.astype(jnp.int32)


_IN_SPECS = [
    pl.BlockSpec((R, D), lambda i: (i, 0)),
    pl.BlockSpec((N, D), lambda i: (0, 0)),
    pl.BlockSpec((K, H1), lambda i: (0, 0)),
    pl.BlockSpec((1, H1), lambda i: (0, 0)),
    pl.BlockSpec((H1, H2), lambda i: (0, 0)),
    pl.BlockSpec((1, H2), lambda i: (0, 0)),
    pl.BlockSpec((H2, 1), lambda i: (0, 0)),
    pl.BlockSpec((1, 1), lambda i: (0, 0)),
]


def _vertex_filts(pts, W1, b1, W2, b2, w3col, b3):
    grid = N // R
    args = (pts, pts, W1, b1.reshape(1, H1), W2, b2.reshape(1, H2),
            w3col, b3)
    out_fast, flags = pl.pallas_call(
        _vertex_body_fast,
        grid=(grid,),
        in_specs=_IN_SPECS,
        out_specs=[pl.BlockSpec((1, 1, R), lambda i: (i, 0, 0)),
                   pl.BlockSpec((1, 1, R), lambda i: (i, 0, 0))],
        out_shape=[jax.ShapeDtypeStruct((grid, 1, R), jnp.float32),
                   jax.ShapeDtypeStruct((grid, 1, R), jnp.int32)],
    )(*args)

    def _slow():
        out = pl.pallas_call(
            _vertex_body_exact,
            grid=(grid,),
            in_specs=_IN_SPECS,
            out_specs=pl.BlockSpec((1, 1, R), lambda i: (i, 0, 0)),
            out_shape=jax.ShapeDtypeStruct((grid, 1, R), jnp.float32),
        )(*args)
        return out.reshape(N)

    return lax.cond(jnp.any(flags > 0), _slow, lambda: out_fast.reshape(N))


def _sqrt_sc(x):
    xi = lax.bitcast_convert_type(x, jnp.int32)
    yi = jnp.int32(0x1FBD1DF5) + (xi >> 1)
    y = lax.bitcast_convert_type(yi, jnp.float32)
    for _ in range(3):
        y = 0.5 * (y + x / y)
    return y


def _edge_body(ptsf_hbm, f_hbm, u_hbm, v_hbm, out_hbm,
               u_v, v_v, iu_v, iv_v, pu_v, pv_v, acc_v, fu_v, fv_v, out_v,
               sem):
    wid = lax.axis_index("s") * 2 + lax.axis_index("c")
    base = wid * EPW
    steps = EPW // 16
    pltpu.sync_copy(u_hbm.at[pl.ds(base, EPW)], u_v)
    pltpu.sync_copy(v_hbm.at[pl.ds(base, EPW)], v_v)
    cu = pltpu.async_copy(f_hbm.at[u_v], fu_v, sem)
    cv = pltpu.async_copy(f_hbm.at[v_v], fv_v, sem)

    def init(i, c):
        sl = pl.ds(i * 16, 16)
        acc_v[sl] = jnp.zeros((16,), jnp.float32)
        iu_v[sl] = u_v[sl] * D
        iv_v[sl] = v_v[sl] * D
        return c

    lax.fori_loop(0, steps, init, 0)
    cu.wait()
    cv.wait()
    for d in range(D):
        gu = pltpu.async_copy(ptsf_hbm.at[iu_v], pu_v, sem)
        gv = pltpu.async_copy(ptsf_hbm.at[iv_v], pv_v, sem)
        gu.wait()
        gv.wait()

        def accd(i, c):
            sl = pl.ds(i * 16, 16)
            df = pu_v[sl] - pv_v[sl]
            acc_v[sl] = acc_v[sl] + df * df
            iu_v[sl] = iu_v[sl] + 1
            iv_v[sl] = iv_v[sl] + 1
            return c

        lax.fori_loop(0, steps, accd, 0)

    def fin(i, c):
        sl = pl.ds(i * 16, 16)
        fm = jnp.maximum(fu_v[sl], fv_v[sl])
        out_v[sl] = _sqrt_sc(jnp.maximum(acc_v[sl], 1e-12)) + fm
        return c

    lax.fori_loop(0, steps, fin, 0)
    pltpu.sync_copy(out_v, out_hbm.at[pl.ds(base, EPW)])


def _edge_filts(ptsf, vfilts, u_pad, v_pad):
    mesh = plsc.VectorSubcoreMesh(core_axis_name="c", subcore_axis_name="s")
    call = pl.kernel(
        _edge_body,
        mesh=mesh,
        out_type=jax.ShapeDtypeStruct((EPAD,), jnp.float32),
        scratch_types=[
            pltpu.VMEM((EPW,), jnp.int32),
            pltpu.VMEM((EPW,), jnp.int32),
            pltpu.VMEM((EPW,), jnp.int32),
            pltpu.VMEM((EPW,), jnp.int32),
            pltpu.VMEM((EPW,), jnp.float32),
            pltpu.VMEM((EPW,), jnp.float32),
            pltpu.VMEM((EPW,), jnp.float32),
            pltpu.VMEM((EPW,), jnp.float32),
            pltpu.VMEM((EPW,), jnp.float32),
            pltpu.VMEM((EPW,), jnp.float32),
            pltpu.SemaphoreType.DMA,
        ],
    )
    return call(ptsf, vfilts, u_pad, v_pad)


def kernel(pts, edges, W1, b1, W2, b2, W3, b3):
    edges = edges.astype(jnp.int32)
    vfilts = _vertex_filts(pts, W1, b1, W2, b2, W3.reshape(H2, 1),
                           b3.reshape(1, 1))
    pad = jnp.zeros((EPAD - E,), jnp.int32)
    u_pad = jnp.concatenate([edges[:, 0], pad])
    v_pad = jnp.concatenate([edges[:, 1], pad])
    efilts = _edge_filts(pts.reshape(-1), vfilts, u_pad, v_pad)
    return vfilts, efilts[:E]
